# baseline (speedup 1.0000x reference)
import functools

import jax
import jax.numpy as jnp
from jax.experimental import pallas as pl

_BLOCK = 2000


def _mlp_head_kernel(x_ref, w1_ref, b1_ref, wh_ref, bh_ref, wv_ref, bv_ref,
                     a_ref, v_ref):
    acc = jnp.dot(x_ref[:, 0, :], w1_ref[0],
                  preferred_element_type=jnp.float32)
    acc += jnp.dot(x_ref[:, 1, :], w1_ref[1],
                   preferred_element_type=jnp.float32)
    acc += jnp.dot(x_ref[:, 2, :], w1_ref[2],
                   preferred_element_type=jnp.float32)
    y = jnp.tanh(acc + b1_ref[...])
    logits = (jnp.dot(y, wh_ref[...], preferred_element_type=jnp.float32)
              + bh_ref[...])
    m = jnp.max(logits, axis=-1, keepdims=True)
    s = logits - m
    lse = jnp.log(jnp.sum(jnp.exp(s), axis=-1, keepdims=True))
    a_ref[...] = s - lse
    v_ref[...] = (jnp.dot(y, wv_ref[...], preferred_element_type=jnp.float32)
                  + bv_ref[...])


@functools.partial(jax.jit, static_argnames=())
def kernel(after_comm, W1, b1, Wh, bh, Wv, bv):
    n, k, hid = after_comm.shape
    n_act = Wh.shape[1]
    b = _BLOCK
    grid = (n // b,)

    a, v = pl.pallas_call(
        _mlp_head_kernel,
        grid=grid,
        in_specs=[
            pl.BlockSpec((b, k, hid), lambda i: (i, 0, 0)),
            pl.BlockSpec((k, hid, hid), lambda i: (0, 0, 0)),
            pl.BlockSpec((1, hid), lambda i: (0, 0)),
            pl.BlockSpec((hid, n_act), lambda i: (0, 0)),
            pl.BlockSpec((1, n_act), lambda i: (0, 0)),
            pl.BlockSpec((hid, 1), lambda i: (0, 0)),
            pl.BlockSpec((1, 1), lambda i: (0, 0)),
        ],
        out_specs=[
            pl.BlockSpec((b, n_act), lambda i: (i, 0)),
            pl.BlockSpec((b, 1), lambda i: (i, 0)),
        ],
        out_shape=[
            jax.ShapeDtypeStruct((n, n_act), jnp.float32),
            jax.ShapeDtypeStruct((n, 1), jnp.float32),
        ],
    )(after_comm, W1.reshape(k, hid, hid), b1.reshape(1, hid),
      Wh, bh.reshape(1, n_act), Wv, bv.reshape(1, 1))
    return (a, v)


# speedup vs baseline: 1.0002x; 1.0002x over previous
import jax
import jax.numpy as jnp
from jax.experimental import pallas as pl
from jax.experimental.pallas import tpu as pltpu

_BLOCK = 2000


def _mlp_head_kernel(x_ref, w1_ref, b1_ref, wh_ref, bh_ref,
                     wv_ref, bv_ref, a_ref, v_ref):
    acc = jnp.dot(x_ref[:, 0, :], w1_ref[0], preferred_element_type=jnp.float32)
    acc += jnp.dot(x_ref[:, 1, :], w1_ref[1], preferred_element_type=jnp.float32)
    acc += jnp.dot(x_ref[:, 2, :], w1_ref[2], preferred_element_type=jnp.float32)
    y = jnp.tanh(acc + b1_ref[...])
    logits = (jnp.dot(y, wh_ref[...], preferred_element_type=jnp.float32)
              + bh_ref[...])
    m = jnp.max(logits, axis=-1, keepdims=True)
    s = logits - m
    lse = jnp.log(jnp.sum(jnp.exp(s), axis=-1, keepdims=True))
    a_ref[...] = s - lse
    v_ref[...] = (jnp.dot(y, wv_ref[...], preferred_element_type=jnp.float32)
                  + bv_ref[...])


@jax.jit
def kernel(after_comm, W1, b1, Wh, bh, Wv, bv):
    n, k, hid = after_comm.shape
    n_act = Wh.shape[1]
    b = _BLOCK
    grid = (n // b,)

    a, v = pl.pallas_call(
        _mlp_head_kernel,
        grid=grid,
        in_specs=[
            pl.BlockSpec((b, k, hid), lambda i: (i, 0, 0)),
            pl.BlockSpec((k, hid, hid), lambda i: (0, 0, 0)),
            pl.BlockSpec((1, hid), lambda i: (0, 0)),
            pl.BlockSpec((hid, n_act), lambda i: (0, 0)),
            pl.BlockSpec((1, n_act), lambda i: (0, 0)),
            pl.BlockSpec((hid, 1), lambda i: (0, 0)),
            pl.BlockSpec((1, 1), lambda i: (0, 0)),
        ],
        out_specs=[
            pl.BlockSpec((b, n_act), lambda i: (i, 0)),
            pl.BlockSpec((b, 1), lambda i: (i, 0)),
        ],
        out_shape=[
            jax.ShapeDtypeStruct((n, n_act), jnp.float32),
            jax.ShapeDtypeStruct((n, 1), jnp.float32),
        ],
        compiler_params=pltpu.CompilerParams(
            dimension_semantics=("parallel",),
        ),
    )(after_comm, W1.reshape(k, hid, hid),
      b1.reshape(1, hid), Wh, bh.reshape(1, n_act), Wv, bv.reshape(1, 1))
    return (a, v)


# speedup vs baseline: 1.0307x; 1.0305x over previous
import jax
import jax.numpy as jnp
from jax.experimental import pallas as pl
from jax.experimental.pallas import tpu as pltpu

_BLOCK = 2000
_STREAMS = 2


def _mlp_head_kernel(*refs):
    x_refs = refs[:_STREAMS]
    w1_ref, b1_ref, wh_ref, bh_ref, wv_ref, bv_ref = refs[_STREAMS:_STREAMS + 6]
    a_ref, v_ref = refs[_STREAMS + 6:]
    b = _BLOCK
    for s, x_ref in enumerate(x_refs):
        acc = jnp.dot(x_ref[:, 0, :], w1_ref[0],
                      preferred_element_type=jnp.float32)
        acc += jnp.dot(x_ref[:, 1, :], w1_ref[1],
                       preferred_element_type=jnp.float32)
        acc += jnp.dot(x_ref[:, 2, :], w1_ref[2],
                       preferred_element_type=jnp.float32)
        y = jnp.tanh(acc + b1_ref[...])
        logits = (jnp.dot(y, wh_ref[...], preferred_element_type=jnp.float32)
                  + bh_ref[...])
        m = jnp.max(logits, axis=-1, keepdims=True)
        sh = logits - m
        lse = jnp.log(jnp.sum(jnp.exp(sh), axis=-1, keepdims=True))
        a_ref[pl.ds(s * b, b), :] = sh - lse
        v_ref[pl.ds(s * b, b), :] = (
            jnp.dot(y, wv_ref[...], preferred_element_type=jnp.float32)
            + bv_ref[...])


@jax.jit
def kernel(after_comm, W1, b1, Wh, bh, Wv, bv):
    n, k, hid = after_comm.shape
    n_act = Wh.shape[1]
    b = _BLOCK
    ns = _STREAMS
    grid = (n // (b * ns),)

    def x_spec(s):
        return pl.BlockSpec((b, k, hid), lambda i, s=s: (ns * i + s, 0, 0))

    a, v = pl.pallas_call(
        _mlp_head_kernel,
        grid=grid,
        in_specs=[x_spec(s) for s in range(ns)] + [
            pl.BlockSpec((k, hid, hid), lambda i: (0, 0, 0)),
            pl.BlockSpec((1, hid), lambda i: (0, 0)),
            pl.BlockSpec((hid, n_act), lambda i: (0, 0)),
            pl.BlockSpec((1, n_act), lambda i: (0, 0)),
            pl.BlockSpec((hid, 1), lambda i: (0, 0)),
            pl.BlockSpec((1, 1), lambda i: (0, 0)),
        ],
        out_specs=[
            pl.BlockSpec((ns * b, n_act), lambda i: (i, 0)),
            pl.BlockSpec((ns * b, 1), lambda i: (i, 0)),
        ],
        out_shape=[
            jax.ShapeDtypeStruct((n, n_act), jnp.float32),
            jax.ShapeDtypeStruct((n, 1), jnp.float32),
        ],
        compiler_params=pltpu.CompilerParams(
            dimension_semantics=("arbitrary",),
        ),
    )(after_comm, after_comm, W1.reshape(k, hid, hid), b1.reshape(1, hid),
      Wh, bh.reshape(1, n_act), Wv, bv.reshape(1, 1))
    return (a, v)
